# baseline (speedup 1.0000x reference)
import functools
import jax
import jax.numpy as jnp
from jax import lax
from jax.experimental import pallas as pl
from jax.experimental.pallas import tpu as pltpu
from jax.experimental.pallas import tpu_sc as plsc

N = 10000
E = 320000
D = 128
DE = 16
NG = 256
INNER = 512
DEPTH = 3

NC, NS = 2, 16
NW = NC * NS
CH = 128
NP = 10240
EPW = 10240
EPAD = NW * EPW
RT = NP // NS
BN = 512
GRID_N = NP // BN


def _sc_mesh():
    return plsc.VectorSubcoreMesh(core_axis_name="c", subcore_axis_name="s",
                                  num_cores=NC, num_subcores=NS)


NCH = EPW // CH


SCH = NCH // 2

CG = CH // 2

NCH0 = 120
NCH1 = 40
NCHMAX = max(NCH0, NCH1)


def _make_spmm_body(nch0, nch1):
    schmax = max(nch0, nch1) // 2

    def body(idx_hbm, x_hbm, zero_hbm, out_hbm,
             idx, r0, r1, acc, s0, s1, s2, s3):
        cid = lax.axis_index("c")
        sid = lax.axis_index("s")
        wid = cid * NS + sid
        sch = jnp.where(cid == 0, nch0 // 2, nch1 // 2)
        schg = 2 * sch
        pltpu.sync_copy(zero_hbm, acc.at[pl.ds(sid * RT, RT)])
        plsc.subcore_barrier()

        rows = (r0, r1)
        sems = (s0, s1, s2, s3)

        def half(k):
            return rows[k // 2].at[pl.ds((k % 2) * CG, CG)]

        def fire(g, k):
            src = idx.at[2 * (g // 2), pl.ds((k % 2) * CG, CG)]
            pltpu.async_copy(x_hbm.at[src], half(k), sems[k])

        def wait(k):
            pltpu.make_async_copy(x_hbm.at[idx.at[0, pl.ds(0, CG)]],
                                  half(k), sems[k]).wait()

        def scat(b, r):
            pltpu.sync_copy(rows[r], acc.at[idx.at[2 * b + 1]], add=True)

        for st in range(2):
            pltpu.sync_copy(
                idx_hbm.at[wid, pl.ds(st * 2 * sch, 2 * schmax)], idx)
            fire(0, 0)
            fire(1, 1)

            def quad(i, carry):
                c = 4 * i
                for j in range(4):
                    wait(j)
                    nxt = c + j + 2

                    @pl.when(nxt < schg)
                    def _():
                        fire(nxt, (j + 2) % 4)

                    if j % 2 == 1:
                        scat((c + j) // 2, j // 2)
                return carry

            lax.fori_loop(0, schg // 4, quad, 0)

        plsc.subcore_barrier()
        pltpu.sync_copy(acc.at[pl.ds(sid * RT, RT)],
                        out_hbm.at[cid, pl.ds(sid * RT, RT)])

    return body


@functools.lru_cache(maxsize=None)
def _spmm_kernel(nch0, nch1):
    schmax = max(nch0, nch1) // 2
    return pl.kernel(
        _make_spmm_body(nch0, nch1),
        out_type=jax.ShapeDtypeStruct((NC, NP, D), jnp.float32),
        mesh=_sc_mesh(),
        scratch_types=[
            pltpu.VMEM((2 * schmax, CH), jnp.int32),
            pltpu.VMEM((CH, D), jnp.float32),
            pltpu.VMEM((CH, D), jnp.float32),
            pltpu.VMEM_SHARED((NP, D), jnp.float32),
            pltpu.SemaphoreType.DMA,
            pltpu.SemaphoreType.DMA,
            pltpu.SemaphoreType.DMA,
            pltpu.SemaphoreType.DMA,
        ],
    )


def _spmm(idx_asym, x_pad, zeroD, nch0=NCH0, nch1=NCH1):
    return _spmm_kernel(nch0, nch1)(idx_asym, x_pad, zeroD)


def _pack_idx(src, dst, nch0, nch1):
    nmax = max(nch0, nch1)
    e0 = NS * nch0 * CH

    def part(arr, fill):
        a0 = arr[:e0].reshape(NS, nch0, CH)
        a1 = arr[e0:].reshape(NS, nch1, CH)
        a0 = jnp.pad(a0, ((0, 0), (0, nmax - nch0), (0, 0)),
                     constant_values=fill)
        a1 = jnp.pad(a1, ((0, 0), (0, nmax - nch1), (0, 0)),
                     constant_values=fill)
        return jnp.concatenate([a0, a1], axis=0)

    s = part(src, 0)
    d = part(dst, N)
    return jnp.stack([s, d], axis=2).reshape(NW, 2 * nmax, CH)


def _eagg_body(idx_hbm, m_hbm, zero_hbm, out_hbm,
               idx, rows0, rows1, acc, sem0, sem1):
    cid = lax.axis_index("c")
    sid = lax.axis_index("s")
    wid = cid * NS + sid
    pltpu.sync_copy(zero_hbm, acc.at[pl.ds(sid * RT, RT)])
    plsc.subcore_barrier()

    for st in range(2):
        pltpu.sync_copy(idx_hbm.at[wid, pl.ds(st * 2 * SCH, 2 * SCH)], idx)
        base = wid * EPW + st * SCH * CH
        pltpu.async_copy(m_hbm.at[pl.ds(base, CH)], rows0, sem0)

        def pair(i, carry):
            c0 = 2 * i
            c1 = 2 * i + 1
            c2 = 2 * i + 2
            pltpu.make_async_copy(m_hbm.at[pl.ds(base, CH)], rows0,
                                  sem0).wait()
            pltpu.async_copy(m_hbm.at[pl.ds(base + c1 * CH, CH)], rows1, sem1)
            pltpu.sync_copy(rows0, acc.at[idx.at[2 * c0 + 1]], add=True)
            pltpu.make_async_copy(m_hbm.at[pl.ds(base, CH)], rows1,
                                  sem1).wait()

            @pl.when(c2 < SCH)
            def _():
                pltpu.async_copy(m_hbm.at[pl.ds(base + c2 * CH, CH)],
                                 rows0, sem0)

            pltpu.sync_copy(rows1, acc.at[idx.at[2 * c1 + 1]], add=True)
            return carry

        lax.fori_loop(0, SCH // 2, pair, 0)

    plsc.subcore_barrier()
    pltpu.sync_copy(acc.at[pl.ds(sid * RT, RT)],
                    out_hbm.at[cid, pl.ds(sid * RT, RT)])


@functools.lru_cache(maxsize=None)
def _eagg_kernel():
    return pl.kernel(
        _eagg_body,
        out_type=jax.ShapeDtypeStruct((NC, NP, D), jnp.float32),
        mesh=_sc_mesh(),
        scratch_types=[
            pltpu.VMEM((2 * SCH, CH), jnp.int32),
            pltpu.VMEM((CH, D), jnp.float32),
            pltpu.VMEM((CH, D), jnp.float32),
            pltpu.VMEM_SHARED((NP, D), jnp.float32),
            pltpu.SemaphoreType.DMA,
            pltpu.SemaphoreType.DMA,
        ],
    )


def _eagg(idx_all, m, zeroD):
    return _eagg_kernel()(idx_all, m, zeroD)


BE = 8192


def _edgemm_body(ea_ref, wet_ref, o_ref):
    o_ref[...] = jnp.dot(ea_ref[...], wet_ref[...],
                         preferred_element_type=jnp.float32,
                         precision=lax.Precision.HIGHEST)


def _edgemm(ea, wet):
    return pl.pallas_call(
        _edgemm_body,
        grid=(EPAD // BE,),
        in_specs=[
            pl.BlockSpec((BE, DE), lambda i: (i, 0)),
            pl.BlockSpec((DE, D), lambda i: (0, 0)),
        ],
        out_specs=pl.BlockSpec((BE, D), lambda i: (i, 0)),
        out_shape=jax.ShapeDtypeStruct((EPAD, D), jnp.float32),
    )(ea, wet)


def _onehot_body(b_ref, o_ref):
    g = lax.broadcasted_iota(jnp.int32, (BN, NG), 1)
    o_ref[...] = jnp.where(b_ref[...] == g, 1.0, 0.0).astype(jnp.float32)


def _onehot(batch2d):
    return pl.pallas_call(
        _onehot_body,
        grid=(GRID_N,),
        in_specs=[pl.BlockSpec((BN, 1), lambda i: (i, 0))],
        out_specs=pl.BlockSpec((BN, NG), lambda i: (i, 0)),
        out_shape=jax.ShapeDtypeStruct((NP, NG), jnp.float32),
    )(batch2d)


def _softpool_body(x_ref, wt_ref, b_ref, oh_ref, pool_ref):
    z = jnp.dot(x_ref[...], wt_ref[...], preferred_element_type=jnp.float32, precision=lax.Precision.HIGHEST)
    z = z + b_ref[...]
    z = z - jnp.max(z, axis=1, keepdims=True)
    ez = jnp.exp(z)
    afp = ez / jnp.sum(ez, axis=1, keepdims=True)
    contrib = lax.dot_general(oh_ref[...], afp, (((0,), (0,)), ((), ())),
                              preferred_element_type=jnp.float32)

    @pl.when(pl.program_id(0) == 0)
    def _():
        pool_ref[...] = jnp.zeros_like(pool_ref)

    pool_ref[...] += contrib


def _softpool(x_pad, wot, b2d, onehot):
    return pl.pallas_call(
        _softpool_body,
        grid=(GRID_N,),
        in_specs=[
            pl.BlockSpec((BN, D), lambda i: (i, 0)),
            pl.BlockSpec((D, INNER), lambda i: (0, 0)),
            pl.BlockSpec((1, INNER), lambda i: (0, 0)),
            pl.BlockSpec((BN, NG), lambda i: (i, 0)),
        ],
        out_specs=pl.BlockSpec((NG, INNER), lambda i: (0, 0)),
        out_shape=jax.ShapeDtypeStruct((NG, INNER), jnp.float32),
    )(x_pad, wot, b2d, onehot)


def _fused_body(y_ref, x_ref, wxt_ref, c_ref, wot_ref, wob_ref, oh_ref,
                xo_ref, pool_ref):
    s = y_ref[0] + y_ref[1] + x_ref[...]
    x2 = jnp.dot(s, wxt_ref[...], preferred_element_type=jnp.float32,
                 precision=lax.Precision.HIGHEST) + c_ref[...]
    xo_ref[...] = x2
    z = jnp.dot(x2, wot_ref[...], preferred_element_type=jnp.float32,
                precision=lax.Precision.HIGHEST)
    z = z + wob_ref[...]
    z = z - jnp.max(z, axis=1, keepdims=True)
    ez = jnp.exp(z)
    afp = ez / jnp.sum(ez, axis=1, keepdims=True)
    contrib = lax.dot_general(oh_ref[...], afp, (((0,), (0,)), ((), ())),
                              preferred_element_type=jnp.float32)

    @pl.when(pl.program_id(0) == 0)
    def _():
        pool_ref[...] = jnp.zeros_like(pool_ref)

    pool_ref[...] += contrib


def _fused(y2, x_pad, wxt, cmat, wot, wob, onehot):
    return pl.pallas_call(
        _fused_body,
        grid=(GRID_N,),
        in_specs=[
            pl.BlockSpec((2, BN, D), lambda i: (0, i, 0)),
            pl.BlockSpec((BN, D), lambda i: (i, 0)),
            pl.BlockSpec((D, D), lambda i: (0, 0)),
            pl.BlockSpec((BN, D), lambda i: (i, 0)),
            pl.BlockSpec((D, INNER), lambda i: (0, 0)),
            pl.BlockSpec((1, INNER), lambda i: (0, 0)),
            pl.BlockSpec((BN, NG), lambda i: (i, 0)),
        ],
        out_specs=[
            pl.BlockSpec((BN, D), lambda i: (i, 0)),
            pl.BlockSpec((NG, INNER), lambda i: (0, 0)),
        ],
        out_shape=[
            jax.ShapeDtypeStruct((NP, D), jnp.float32),
            jax.ShapeDtypeStruct((NG, INNER), jnp.float32),
        ],
    )(y2, x_pad, wxt, cmat, wot, wob, onehot)


def _cmat_body(aggm_ref, b_ref, o_ref):
    o_ref[...] = aggm_ref[0] + aggm_ref[1] + b_ref[...]


def _cmat(aggm, b2d):
    return pl.pallas_call(
        _cmat_body,
        grid=(GRID_N,),
        in_specs=[
            pl.BlockSpec((2, BN, D), lambda i: (0, i, 0)),
            pl.BlockSpec((1, D), lambda i: (0, 0)),
        ],
        out_specs=pl.BlockSpec((BN, D), lambda i: (i, 0)),
        out_shape=jax.ShapeDtypeStruct((NP, D), jnp.float32),
    )(aggm, b2d)


def _head_body(p0, p1, p2, p3, l1t, l1b, l2t, l2b, o_ref):
    ov = p0[...] + p1[...] + p2[...] + p3[...]
    h = jnp.dot(ov, l1t[...], preferred_element_type=jnp.float32, precision=lax.Precision.HIGHEST) + l1b[...]
    t = jnp.dot(h, l2t[...], preferred_element_type=jnp.float32, precision=lax.Precision.HIGHEST) + l2b[...]
    o_ref[...] = jax.nn.sigmoid(t)


def _head(p0, p1, p2, p3, l1t, l1b, l2t, l2b):
    return pl.pallas_call(
        _head_body,
        out_shape=jax.ShapeDtypeStruct((NG, 1), jnp.float32),
    )(p0, p1, p2, p3, l1t, l1b, l2t, l2b)


def kernel(x, edge_index, edge_attr, batch,
           W_out_w, W_out_b, W_in_w, W_in_b,
           lin1_w, lin1_b, lin2_w, lin2_b):
    f32 = jnp.float32
    x_pad = jnp.pad(x, ((0, NP - N), (0, 0)))
    src = jnp.pad(edge_index[0], (0, EPAD - E))
    dst = jnp.pad(edge_index[1], (0, EPAD - E), constant_values=N)
    idx_all = jnp.stack(
        [src.reshape(NW, NCH, CH), dst.reshape(NW, NCH, CH)], axis=2
    ).reshape(NW, 2 * NCH, CH)
    idx_asym = _pack_idx(src, dst, NCH0, NCH1)
    ea = jnp.pad(edge_attr, ((0, EPAD - E), (0, 0)))
    batch2d = jnp.pad(batch, (0, NP - N), constant_values=-1)[:, None]

    wot = W_out_w.T
    wob = W_out_b[None, :]
    wxt = W_in_w[:, :D].T
    wet = W_in_w[:, D:].T
    wib = W_in_b[None, :]
    l1t = lin1_w.T
    l1b = lin1_b[None, :]
    l2t = lin2_w.T
    l2b = lin2_b[None, :]

    zeroD = jnp.zeros((RT, D), f32)

    onehot = _onehot(batch2d)
    m = _edgemm(ea, wet)
    aggm = _eagg(idx_all, m, zeroD)
    cmat = _cmat(aggm, wib)

    pools = [_softpool(x_pad, wot, wob, onehot)]
    for i in range(DEPTH):
        y2 = _spmm(idx_asym, x_pad, zeroD)
        x_pad, pool = _fused(y2, x_pad, wxt, cmat, wot, wob, onehot)
        pools.append(pool)

    return _head(pools[0], pools[1], pools[2], pools[3],
                 l1t, l1b, l2t, l2b)


# speedup vs baseline: 1.0042x; 1.0042x over previous
import functools
import jax
import jax.numpy as jnp
from jax import lax
from jax.experimental import pallas as pl
from jax.experimental.pallas import tpu as pltpu
from jax.experimental.pallas import tpu_sc as plsc

N = 10000
E = 320000
D = 128
DE = 16
NG = 256
INNER = 512
DEPTH = 3

NC, NS = 2, 16
NW = NC * NS
CH = 128
NP = 10240
EPW = 10240
EPAD = NW * EPW
RT = NP // NS
BN = 512
GRID_N = NP // BN


def _sc_mesh():
    return plsc.VectorSubcoreMesh(core_axis_name="c", subcore_axis_name="s",
                                  num_cores=NC, num_subcores=NS)


NCH = EPW // CH


SCH = NCH // 2

CG = CH // 2

NCH0 = 104
NCH1 = 56
NCHMAX = max(NCH0, NCH1)


def _make_spmm_body(nch0, nch1):
    schmax = max(nch0, nch1) // 2

    def body(idx_hbm, x_hbm, zero_hbm, out_hbm,
             idx, r0, r1, acc, s0, s1, s2, s3):
        cid = lax.axis_index("c")
        sid = lax.axis_index("s")
        wid = cid * NS + sid
        sch = jnp.where(cid == 0, nch0 // 2, nch1 // 2)
        schg = 2 * sch
        pltpu.sync_copy(zero_hbm, acc.at[pl.ds(sid * RT, RT)])
        plsc.subcore_barrier()

        rows = (r0, r1)
        sems = (s0, s1, s2, s3)

        def half(k):
            return rows[k // 2].at[pl.ds((k % 2) * CG, CG)]

        def fire(g, k):
            src = idx.at[2 * (g // 2), pl.ds((k % 2) * CG, CG)]
            pltpu.async_copy(x_hbm.at[src], half(k), sems[k])

        def wait(k):
            pltpu.make_async_copy(x_hbm.at[idx.at[0, pl.ds(0, CG)]],
                                  half(k), sems[k]).wait()

        def scat(b, r):
            pltpu.sync_copy(rows[r], acc.at[idx.at[2 * b + 1]], add=True)

        for st in range(2):
            pltpu.sync_copy(
                idx_hbm.at[wid, pl.ds(st * 2 * sch, 2 * schmax)], idx)
            fire(0, 0)
            fire(1, 1)

            def quad(i, carry):
                c = 4 * i
                for j in range(4):
                    wait(j)
                    nxt = c + j + 2

                    @pl.when(nxt < schg)
                    def _():
                        fire(nxt, (j + 2) % 4)

                    if j % 2 == 1:
                        scat((c + j) // 2, j // 2)
                return carry

            lax.fori_loop(0, schg // 4, quad, 0)

        plsc.subcore_barrier()
        pltpu.sync_copy(acc.at[pl.ds(sid * RT, RT)],
                        out_hbm.at[cid, pl.ds(sid * RT, RT)])

    return body


@functools.lru_cache(maxsize=None)
def _spmm_kernel(nch0, nch1):
    schmax = max(nch0, nch1) // 2
    return pl.kernel(
        _make_spmm_body(nch0, nch1),
        out_type=jax.ShapeDtypeStruct((NC, NP, D), jnp.float32),
        mesh=_sc_mesh(),
        scratch_types=[
            pltpu.VMEM((2 * schmax, CH), jnp.int32),
            pltpu.VMEM((CH, D), jnp.float32),
            pltpu.VMEM((CH, D), jnp.float32),
            pltpu.VMEM_SHARED((NP, D), jnp.float32),
            pltpu.SemaphoreType.DMA,
            pltpu.SemaphoreType.DMA,
            pltpu.SemaphoreType.DMA,
            pltpu.SemaphoreType.DMA,
        ],
    )


def _spmm(idx_asym, x_pad, zeroD, nch0=NCH0, nch1=NCH1):
    return _spmm_kernel(nch0, nch1)(idx_asym, x_pad, zeroD)


def _pack_idx(src, dst, nch0, nch1):
    nmax = max(nch0, nch1)
    e0 = NS * nch0 * CH

    def part(arr, fill):
        a0 = arr[:e0].reshape(NS, nch0, CH)
        a1 = arr[e0:].reshape(NS, nch1, CH)
        a0 = jnp.pad(a0, ((0, 0), (0, nmax - nch0), (0, 0)),
                     constant_values=fill)
        a1 = jnp.pad(a1, ((0, 0), (0, nmax - nch1), (0, 0)),
                     constant_values=fill)
        return jnp.concatenate([a0, a1], axis=0)

    s = part(src, 0)
    d = part(dst, N)
    return jnp.stack([s, d], axis=2).reshape(NW, 2 * nmax, CH)


def _eagg_body(idx_hbm, m_hbm, zero_hbm, out_hbm,
               idx, rows0, rows1, acc, sem0, sem1):
    cid = lax.axis_index("c")
    sid = lax.axis_index("s")
    wid = cid * NS + sid
    pltpu.sync_copy(zero_hbm, acc.at[pl.ds(sid * RT, RT)])
    plsc.subcore_barrier()

    for st in range(2):
        pltpu.sync_copy(idx_hbm.at[wid, pl.ds(st * 2 * SCH, 2 * SCH)], idx)
        base = wid * EPW + st * SCH * CH
        pltpu.async_copy(m_hbm.at[pl.ds(base, CH)], rows0, sem0)

        def pair(i, carry):
            c0 = 2 * i
            c1 = 2 * i + 1
            c2 = 2 * i + 2
            pltpu.make_async_copy(m_hbm.at[pl.ds(base, CH)], rows0,
                                  sem0).wait()
            pltpu.async_copy(m_hbm.at[pl.ds(base + c1 * CH, CH)], rows1, sem1)
            pltpu.sync_copy(rows0, acc.at[idx.at[2 * c0 + 1]], add=True)
            pltpu.make_async_copy(m_hbm.at[pl.ds(base, CH)], rows1,
                                  sem1).wait()

            @pl.when(c2 < SCH)
            def _():
                pltpu.async_copy(m_hbm.at[pl.ds(base + c2 * CH, CH)],
                                 rows0, sem0)

            pltpu.sync_copy(rows1, acc.at[idx.at[2 * c1 + 1]], add=True)
            return carry

        lax.fori_loop(0, SCH // 2, pair, 0)

    plsc.subcore_barrier()
    pltpu.sync_copy(acc.at[pl.ds(sid * RT, RT)],
                    out_hbm.at[cid, pl.ds(sid * RT, RT)])


@functools.lru_cache(maxsize=None)
def _eagg_kernel():
    return pl.kernel(
        _eagg_body,
        out_type=jax.ShapeDtypeStruct((NC, NP, D), jnp.float32),
        mesh=_sc_mesh(),
        scratch_types=[
            pltpu.VMEM((2 * SCH, CH), jnp.int32),
            pltpu.VMEM((CH, D), jnp.float32),
            pltpu.VMEM((CH, D), jnp.float32),
            pltpu.VMEM_SHARED((NP, D), jnp.float32),
            pltpu.SemaphoreType.DMA,
            pltpu.SemaphoreType.DMA,
        ],
    )


def _eagg(idx_all, m, zeroD):
    return _eagg_kernel()(idx_all, m, zeroD)


BE = 8192


def _edgemm_body(ea_ref, wet_ref, o_ref):
    o_ref[...] = jnp.dot(ea_ref[...], wet_ref[...],
                         preferred_element_type=jnp.float32,
                         precision=lax.Precision.HIGHEST)


def _edgemm(ea, wet):
    return pl.pallas_call(
        _edgemm_body,
        grid=(EPAD // BE,),
        in_specs=[
            pl.BlockSpec((BE, DE), lambda i: (i, 0)),
            pl.BlockSpec((DE, D), lambda i: (0, 0)),
        ],
        out_specs=pl.BlockSpec((BE, D), lambda i: (i, 0)),
        out_shape=jax.ShapeDtypeStruct((EPAD, D), jnp.float32),
    )(ea, wet)


def _onehot_body(b_ref, o_ref):
    g = lax.broadcasted_iota(jnp.int32, (BN, NG), 1)
    o_ref[...] = jnp.where(b_ref[...] == g, 1.0, 0.0).astype(jnp.float32)


def _onehot(batch2d):
    return pl.pallas_call(
        _onehot_body,
        grid=(GRID_N,),
        in_specs=[pl.BlockSpec((BN, 1), lambda i: (i, 0))],
        out_specs=pl.BlockSpec((BN, NG), lambda i: (i, 0)),
        out_shape=jax.ShapeDtypeStruct((NP, NG), jnp.float32),
    )(batch2d)


def _softpool_body(x_ref, wt_ref, b_ref, oh_ref, pool_ref):
    z = jnp.dot(x_ref[...], wt_ref[...], preferred_element_type=jnp.float32, precision=lax.Precision.HIGHEST)
    z = z + b_ref[...]
    z = z - jnp.max(z, axis=1, keepdims=True)
    ez = jnp.exp(z)
    afp = ez / jnp.sum(ez, axis=1, keepdims=True)
    contrib = lax.dot_general(oh_ref[...], afp, (((0,), (0,)), ((), ())),
                              preferred_element_type=jnp.float32)

    @pl.when(pl.program_id(0) == 0)
    def _():
        pool_ref[...] = jnp.zeros_like(pool_ref)

    pool_ref[...] += contrib


def _softpool(x_pad, wot, b2d, onehot):
    return pl.pallas_call(
        _softpool_body,
        grid=(GRID_N,),
        in_specs=[
            pl.BlockSpec((BN, D), lambda i: (i, 0)),
            pl.BlockSpec((D, INNER), lambda i: (0, 0)),
            pl.BlockSpec((1, INNER), lambda i: (0, 0)),
            pl.BlockSpec((BN, NG), lambda i: (i, 0)),
        ],
        out_specs=pl.BlockSpec((NG, INNER), lambda i: (0, 0)),
        out_shape=jax.ShapeDtypeStruct((NG, INNER), jnp.float32),
    )(x_pad, wot, b2d, onehot)


def _update_body(y_ref, x_ref, wxt_ref, c_ref, o_ref):
    s = y_ref[0] + y_ref[1] + x_ref[...]
    o_ref[...] = jnp.dot(s, wxt_ref[...],
                         preferred_element_type=jnp.float32, precision=lax.Precision.HIGHEST) + c_ref[...]


def _update(y2, x_pad, wxt, cmat):
    return pl.pallas_call(
        _update_body,
        grid=(GRID_N,),
        in_specs=[
            pl.BlockSpec((2, BN, D), lambda i: (0, i, 0)),
            pl.BlockSpec((BN, D), lambda i: (i, 0)),
            pl.BlockSpec((D, D), lambda i: (0, 0)),
            pl.BlockSpec((BN, D), lambda i: (i, 0)),
        ],
        out_specs=pl.BlockSpec((BN, D), lambda i: (i, 0)),
        out_shape=jax.ShapeDtypeStruct((NP, D), jnp.float32),
    )(y2, x_pad, wxt, cmat)


def _cmat_body(aggm_ref, b_ref, o_ref):
    o_ref[...] = aggm_ref[0] + aggm_ref[1] + b_ref[...]


def _cmat(aggm, b2d):
    return pl.pallas_call(
        _cmat_body,
        grid=(GRID_N,),
        in_specs=[
            pl.BlockSpec((2, BN, D), lambda i: (0, i, 0)),
            pl.BlockSpec((1, D), lambda i: (0, 0)),
        ],
        out_specs=pl.BlockSpec((BN, D), lambda i: (i, 0)),
        out_shape=jax.ShapeDtypeStruct((NP, D), jnp.float32),
    )(aggm, b2d)


def _head_body(p0, p1, p2, p3, l1t, l1b, l2t, l2b, o_ref):
    ov = p0[...] + p1[...] + p2[...] + p3[...]
    h = jnp.dot(ov, l1t[...], preferred_element_type=jnp.float32, precision=lax.Precision.HIGHEST) + l1b[...]
    t = jnp.dot(h, l2t[...], preferred_element_type=jnp.float32, precision=lax.Precision.HIGHEST) + l2b[...]
    o_ref[...] = jax.nn.sigmoid(t)


def _head(p0, p1, p2, p3, l1t, l1b, l2t, l2b):
    return pl.pallas_call(
        _head_body,
        out_shape=jax.ShapeDtypeStruct((NG, 1), jnp.float32),
    )(p0, p1, p2, p3, l1t, l1b, l2t, l2b)


def kernel(x, edge_index, edge_attr, batch,
           W_out_w, W_out_b, W_in_w, W_in_b,
           lin1_w, lin1_b, lin2_w, lin2_b):
    f32 = jnp.float32
    x_pad = jnp.pad(x, ((0, NP - N), (0, 0)))
    src = jnp.pad(edge_index[0], (0, EPAD - E))
    dst = jnp.pad(edge_index[1], (0, EPAD - E), constant_values=N)
    idx_all = jnp.stack(
        [src.reshape(NW, NCH, CH), dst.reshape(NW, NCH, CH)], axis=2
    ).reshape(NW, 2 * NCH, CH)
    idx_asym = _pack_idx(src, dst, NCH0, NCH1)
    ea = jnp.pad(edge_attr, ((0, EPAD - E), (0, 0)))
    batch2d = jnp.pad(batch, (0, NP - N), constant_values=-1)[:, None]

    wot = W_out_w.T
    wob = W_out_b[None, :]
    wxt = W_in_w[:, :D].T
    wet = W_in_w[:, D:].T
    wib = W_in_b[None, :]
    l1t = lin1_w.T
    l1b = lin1_b[None, :]
    l2t = lin2_w.T
    l2b = lin2_b[None, :]

    zeroD = jnp.zeros((RT, D), f32)

    onehot = _onehot(batch2d)
    m = _edgemm(ea, wet)
    aggm = _eagg(idx_all, m, zeroD)
    cmat = _cmat(aggm, wib)

    pools = []
    for i in range(DEPTH + 1):
        y2 = _spmm(idx_asym, x_pad, zeroD) if i < DEPTH else None
        pools.append(_softpool(x_pad, wot, wob, onehot))
        if i < DEPTH:
            x_pad = _update(y2, x_pad, wxt, cmat)

    return _head(pools[0], pools[1], pools[2], pools[3],
                 l1t, l1b, l2t, l2b)


# speedup vs baseline: 1.0161x; 1.0119x over previous
import functools
import jax
import jax.numpy as jnp
from jax import lax
from jax.experimental import pallas as pl
from jax.experimental.pallas import tpu as pltpu
from jax.experimental.pallas import tpu_sc as plsc

N = 10000
E = 320000
D = 128
DE = 16
NG = 256
INNER = 512
DEPTH = 3

NC, NS = 2, 16
NW = NC * NS
CH = 128
NP = 10240
EPW = 10240
EPAD = NW * EPW
RT = NP // NS
BN = 512
GRID_N = NP // BN


def _sc_mesh():
    return plsc.VectorSubcoreMesh(core_axis_name="c", subcore_axis_name="s",
                                  num_cores=NC, num_subcores=NS)


NCH = EPW // CH


SCH = NCH // 2

CG = CH // 2

NCH0 = 120
NCH1 = 40
NCHMAX = max(NCH0, NCH1)


def _make_spmm_body(nch0, nch1):
    schmax = max(nch0, nch1) // 2

    def body(idx_hbm, x_hbm, zero_hbm, out_hbm,
             idx, r0, r1, acc, s0, s1, s2, s3):
        cid = lax.axis_index("c")
        sid = lax.axis_index("s")
        wid = cid * NS + sid
        sch = jnp.where(cid == 0, nch0 // 2, nch1 // 2)
        schg = 2 * sch
        pltpu.sync_copy(zero_hbm, acc.at[pl.ds(sid * RT, RT)])
        plsc.subcore_barrier()

        rows = (r0, r1)
        sems = (s0, s1, s2, s3)

        def half(k):
            return rows[k // 2].at[pl.ds((k % 2) * CG, CG)]

        def fire(g, k):
            src = idx.at[2 * (g // 2), pl.ds((k % 2) * CG, CG)]
            pltpu.async_copy(x_hbm.at[src], half(k), sems[k])

        def wait(k):
            pltpu.make_async_copy(x_hbm.at[idx.at[0, pl.ds(0, CG)]],
                                  half(k), sems[k]).wait()

        def scat(b, r):
            pltpu.sync_copy(rows[r], acc.at[idx.at[2 * b + 1]], add=True)

        for st in range(2):
            pltpu.sync_copy(
                idx_hbm.at[wid, pl.ds(st * 2 * sch, 2 * schmax)], idx)
            fire(0, 0)
            fire(1, 1)

            def quad(i, carry):
                c = 4 * i
                for j in range(4):
                    wait(j)
                    nxt = c + j + 2

                    @pl.when(nxt < schg)
                    def _():
                        fire(nxt, (j + 2) % 4)

                    if j % 2 == 1:
                        scat((c + j) // 2, j // 2)
                return carry

            lax.fori_loop(0, schg // 4, quad, 0)

        plsc.subcore_barrier()
        pltpu.sync_copy(acc.at[pl.ds(sid * RT, RT)],
                        out_hbm.at[cid, pl.ds(sid * RT, RT)])

    return body


@functools.lru_cache(maxsize=None)
def _spmm_kernel(nch0, nch1):
    schmax = max(nch0, nch1) // 2
    return pl.kernel(
        _make_spmm_body(nch0, nch1),
        out_type=jax.ShapeDtypeStruct((NC, NP, D), jnp.float32),
        mesh=_sc_mesh(),
        scratch_types=[
            pltpu.VMEM((2 * schmax, CH), jnp.int32),
            pltpu.VMEM((CH, D), jnp.float32),
            pltpu.VMEM((CH, D), jnp.float32),
            pltpu.VMEM_SHARED((NP, D), jnp.float32),
            pltpu.SemaphoreType.DMA,
            pltpu.SemaphoreType.DMA,
            pltpu.SemaphoreType.DMA,
            pltpu.SemaphoreType.DMA,
        ],
    )


def _spmm(idx_asym, x_pad, zeroD, nch0=NCH0, nch1=NCH1):
    return _spmm_kernel(nch0, nch1)(idx_asym, x_pad, zeroD)


def _pack_idx(src, dst, nch0, nch1):
    nmax = max(nch0, nch1)
    e0 = NS * nch0 * CH

    def part(arr, fill):
        a0 = arr[:e0].reshape(NS, nch0, CH)
        a1 = arr[e0:].reshape(NS, nch1, CH)
        a0 = jnp.pad(a0, ((0, 0), (0, nmax - nch0), (0, 0)),
                     constant_values=fill)
        a1 = jnp.pad(a1, ((0, 0), (0, nmax - nch1), (0, 0)),
                     constant_values=fill)
        return jnp.concatenate([a0, a1], axis=0)

    s = part(src, 0)
    d = part(dst, N)
    return jnp.stack([s, d], axis=2).reshape(NW, 2 * nmax, CH)


def _eagg_body(idx_hbm, m_hbm, zero_hbm, out_hbm,
               idx, rows0, rows1, acc, sem0, sem1):
    cid = lax.axis_index("c")
    sid = lax.axis_index("s")
    wid = cid * NS + sid
    pltpu.sync_copy(zero_hbm, acc.at[pl.ds(sid * RT, RT)])
    plsc.subcore_barrier()

    for st in range(2):
        pltpu.sync_copy(idx_hbm.at[wid, pl.ds(st * 2 * SCH, 2 * SCH)], idx)
        base = wid * EPW + st * SCH * CH
        pltpu.async_copy(m_hbm.at[pl.ds(base, CH)], rows0, sem0)

        def pair(i, carry):
            c0 = 2 * i
            c1 = 2 * i + 1
            c2 = 2 * i + 2
            pltpu.make_async_copy(m_hbm.at[pl.ds(base, CH)], rows0,
                                  sem0).wait()
            pltpu.async_copy(m_hbm.at[pl.ds(base + c1 * CH, CH)], rows1, sem1)
            pltpu.sync_copy(rows0, acc.at[idx.at[2 * c0 + 1]], add=True)
            pltpu.make_async_copy(m_hbm.at[pl.ds(base, CH)], rows1,
                                  sem1).wait()

            @pl.when(c2 < SCH)
            def _():
                pltpu.async_copy(m_hbm.at[pl.ds(base + c2 * CH, CH)],
                                 rows0, sem0)

            pltpu.sync_copy(rows1, acc.at[idx.at[2 * c1 + 1]], add=True)
            return carry

        lax.fori_loop(0, SCH // 2, pair, 0)

    plsc.subcore_barrier()
    pltpu.sync_copy(acc.at[pl.ds(sid * RT, RT)],
                    out_hbm.at[cid, pl.ds(sid * RT, RT)])


@functools.lru_cache(maxsize=None)
def _eagg_kernel():
    return pl.kernel(
        _eagg_body,
        out_type=jax.ShapeDtypeStruct((NC, NP, D), jnp.float32),
        mesh=_sc_mesh(),
        scratch_types=[
            pltpu.VMEM((2 * SCH, CH), jnp.int32),
            pltpu.VMEM((CH, D), jnp.float32),
            pltpu.VMEM((CH, D), jnp.float32),
            pltpu.VMEM_SHARED((NP, D), jnp.float32),
            pltpu.SemaphoreType.DMA,
            pltpu.SemaphoreType.DMA,
        ],
    )


def _eagg(idx_all, m, zeroD):
    return _eagg_kernel()(idx_all, m, zeroD)


BE = 8192


def _edgemm_body(ea_ref, wet_ref, o_ref):
    o_ref[...] = jnp.dot(ea_ref[...], wet_ref[...],
                         preferred_element_type=jnp.float32,
                         precision=lax.Precision.HIGHEST)


def _edgemm(ea, wet):
    return pl.pallas_call(
        _edgemm_body,
        grid=(EPAD // BE,),
        in_specs=[
            pl.BlockSpec((BE, DE), lambda i: (i, 0)),
            pl.BlockSpec((DE, D), lambda i: (0, 0)),
        ],
        out_specs=pl.BlockSpec((BE, D), lambda i: (i, 0)),
        out_shape=jax.ShapeDtypeStruct((EPAD, D), jnp.float32),
    )(ea, wet)


def _onehot_body(b_ref, o_ref):
    g = lax.broadcasted_iota(jnp.int32, (BN, NG), 1)
    o_ref[...] = jnp.where(b_ref[...] == g, 1.0, 0.0).astype(jnp.float32)


def _onehot(batch2d):
    return pl.pallas_call(
        _onehot_body,
        grid=(GRID_N,),
        in_specs=[pl.BlockSpec((BN, 1), lambda i: (i, 0))],
        out_specs=pl.BlockSpec((BN, NG), lambda i: (i, 0)),
        out_shape=jax.ShapeDtypeStruct((NP, NG), jnp.float32),
    )(batch2d)


def _softpool_body(x_ref, wt_ref, b_ref, oh_ref, pool_ref):
    z = jnp.dot(x_ref[...], wt_ref[...], preferred_element_type=jnp.float32, precision=lax.Precision.HIGHEST)
    z = z + b_ref[...]
    z = z - jnp.max(z, axis=1, keepdims=True)
    ez = jnp.exp(z)
    afp = ez / jnp.sum(ez, axis=1, keepdims=True)
    contrib = lax.dot_general(oh_ref[...], afp, (((0,), (0,)), ((), ())),
                              preferred_element_type=jnp.float32)

    @pl.when(pl.program_id(0) == 0)
    def _():
        pool_ref[...] = jnp.zeros_like(pool_ref)

    pool_ref[...] += contrib


def _softpool(x_pad, wot, b2d, onehot):
    return pl.pallas_call(
        _softpool_body,
        grid=(GRID_N,),
        in_specs=[
            pl.BlockSpec((BN, D), lambda i: (i, 0)),
            pl.BlockSpec((D, INNER), lambda i: (0, 0)),
            pl.BlockSpec((1, INNER), lambda i: (0, 0)),
            pl.BlockSpec((BN, NG), lambda i: (i, 0)),
        ],
        out_specs=pl.BlockSpec((NG, INNER), lambda i: (0, 0)),
        out_shape=jax.ShapeDtypeStruct((NG, INNER), jnp.float32),
    )(x_pad, wot, b2d, onehot)


def _update_body(y_ref, x_ref, wxt_ref, c_ref, o_ref):
    s = y_ref[0] + y_ref[1] + x_ref[...]
    o_ref[...] = jnp.dot(s, wxt_ref[...],
                         preferred_element_type=jnp.float32, precision=lax.Precision.HIGHEST) + c_ref[...]


def _update(y2, x_pad, wxt, cmat):
    return pl.pallas_call(
        _update_body,
        grid=(GRID_N,),
        in_specs=[
            pl.BlockSpec((2, BN, D), lambda i: (0, i, 0)),
            pl.BlockSpec((BN, D), lambda i: (i, 0)),
            pl.BlockSpec((D, D), lambda i: (0, 0)),
            pl.BlockSpec((BN, D), lambda i: (i, 0)),
        ],
        out_specs=pl.BlockSpec((BN, D), lambda i: (i, 0)),
        out_shape=jax.ShapeDtypeStruct((NP, D), jnp.float32),
    )(y2, x_pad, wxt, cmat)


def _cmat_body(aggm_ref, b_ref, o_ref):
    o_ref[...] = aggm_ref[0] + aggm_ref[1] + b_ref[...]


def _cmat(aggm, b2d):
    return pl.pallas_call(
        _cmat_body,
        grid=(GRID_N,),
        in_specs=[
            pl.BlockSpec((2, BN, D), lambda i: (0, i, 0)),
            pl.BlockSpec((1, D), lambda i: (0, 0)),
        ],
        out_specs=pl.BlockSpec((BN, D), lambda i: (i, 0)),
        out_shape=jax.ShapeDtypeStruct((NP, D), jnp.float32),
    )(aggm, b2d)


def _head_body(p0, p1, p2, p3, l1t, l1b, l2t, l2b, o_ref):
    ov = p0[...] + p1[...] + p2[...] + p3[...]
    h = jnp.dot(ov, l1t[...], preferred_element_type=jnp.float32, precision=lax.Precision.HIGHEST) + l1b[...]
    t = jnp.dot(h, l2t[...], preferred_element_type=jnp.float32, precision=lax.Precision.HIGHEST) + l2b[...]
    o_ref[...] = jax.nn.sigmoid(t)


def _head(p0, p1, p2, p3, l1t, l1b, l2t, l2b):
    return pl.pallas_call(
        _head_body,
        out_shape=jax.ShapeDtypeStruct((NG, 1), jnp.float32),
    )(p0, p1, p2, p3, l1t, l1b, l2t, l2b)


def kernel(x, edge_index, edge_attr, batch,
           W_out_w, W_out_b, W_in_w, W_in_b,
           lin1_w, lin1_b, lin2_w, lin2_b):
    f32 = jnp.float32
    x_pad = jnp.pad(x, ((0, NP - N), (0, 0)))
    src = jnp.pad(edge_index[0], (0, EPAD - E))
    dst = jnp.pad(edge_index[1], (0, EPAD - E), constant_values=N)
    idx_all = jnp.stack(
        [src.reshape(NW, NCH, CH), dst.reshape(NW, NCH, CH)], axis=2
    ).reshape(NW, 2 * NCH, CH)
    idx_asym = _pack_idx(src, dst, NCH0, NCH1)
    ea = jnp.pad(edge_attr, ((0, EPAD - E), (0, 0)))
    batch2d = jnp.pad(batch, (0, NP - N), constant_values=-1)[:, None]

    wot = W_out_w.T
    wob = W_out_b[None, :]
    wxt = W_in_w[:, :D].T
    wet = W_in_w[:, D:].T
    wib = W_in_b[None, :]
    l1t = lin1_w.T
    l1b = lin1_b[None, :]
    l2t = lin2_w.T
    l2b = lin2_b[None, :]

    zeroD = jnp.zeros((RT, D), f32)

    onehot = _onehot(batch2d)
    m = _edgemm(ea, wet)
    aggm = _eagg(idx_all, m, zeroD)
    cmat = _cmat(aggm, wib)

    pools = []
    for i in range(DEPTH + 1):
        y2 = _spmm(idx_asym, x_pad, zeroD) if i < DEPTH else None
        pools.append(_softpool(x_pad, wot, wob, onehot))
        if i < DEPTH:
            x_pad = _update(y2, x_pad, wxt, cmat)

    return _head(pools[0], pools[1], pools[2], pools[3],
                 l1t, l1b, l2t, l2b)
